# baseline (speedup 1.0000x reference)
import jax
import jax.numpy as jnp
from jax.experimental import pallas as pl

N, D = 16384, 512
BLOCK_ROWS = 1024
GRID = N // BLOCK_ROWS


def _sse_block(pred_ref, true_ref, out_ref):
    i = pl.program_id(0)

    @pl.when(i == 0)
    def _():
        out_ref[...] = jnp.zeros_like(out_ref)

    d = pred_ref[...] - true_ref[...]
    out_ref[...] += jnp.sum(d * d).reshape(1, 1)


def kernel(y_pred, y_true):
    total = pl.pallas_call(
        _sse_block,
        grid=(GRID,),
        in_specs=[
            pl.BlockSpec((BLOCK_ROWS, D), lambda i: (i, 0)),
            pl.BlockSpec((BLOCK_ROWS, D), lambda i: (i, 0)),
        ],
        out_specs=pl.BlockSpec((1, 1), lambda i: (0, 0)),
        out_shape=jax.ShapeDtypeStruct((1, 1), jnp.float32),
    )(y_pred, y_true)
    return total[0, 0] / N


# speedup vs baseline: 1.1262x; 1.1262x over previous
import jax
import jax.numpy as jnp
from jax.experimental import pallas as pl
from jax.experimental.pallas import tpu as pltpu

N, D = 16384, 512
BLOCK_ROWS = 2048
GRID = N // BLOCK_ROWS


def _sse_block(pred_ref, true_ref, out_ref, acc_ref):
    i = pl.program_id(0)

    @pl.when(i == 0)
    def _():
        acc_ref[...] = jnp.zeros_like(acc_ref)

    d = pred_ref[...] - true_ref[...]
    acc_ref[...] += jnp.sum(d * d, axis=0, keepdims=True)

    @pl.when(i == GRID - 1)
    def _():
        out_ref[...] = jnp.sum(acc_ref[...]).reshape(1, 1)


def kernel(y_pred, y_true):
    total = pl.pallas_call(
        _sse_block,
        grid=(GRID,),
        in_specs=[
            pl.BlockSpec((BLOCK_ROWS, D), lambda i: (i, 0)),
            pl.BlockSpec((BLOCK_ROWS, D), lambda i: (i, 0)),
        ],
        out_specs=pl.BlockSpec((1, 1), lambda i: (0, 0)),
        out_shape=jax.ShapeDtypeStruct((1, 1), jnp.float32),
        scratch_shapes=[pltpu.VMEM((1, D), jnp.float32)],
    )(y_pred, y_true)
    return total[0, 0] / N
